# baseline (speedup 1.0000x reference)
import functools

import jax
import jax.numpy as jnp
from jax import lax
from jax.experimental import pallas as pl
from jax.experimental.pallas import tpu as pltpu
from jax.experimental.pallas import tpu_sc as plsc

N, E, D, G = 10000, 320000, 128, 128
NC, NS = 2, 16
NW = NC * NS
CHUNK = 128
CPW = 80
GS = 16
NG = CPW // GS
EPAD = NW * CPW * CHUNK
NPAD = 10240
RPS = NPAD // NS

BLK = 2048
NBLK = NPAD // BLK


def _agg_body(h_hbm, src_hbm, dst_hbm, out_hbm,
              acc, src_v, dst_v, rows_v, sem0, sem1):
    c = lax.axis_index("c")
    s = lax.axis_index("s")
    w = c * NS + s
    sems = (sem0, sem1)

    pltpu.sync_copy(h_hbm.at[pl.ds(s * RPS, RPS)], acc.at[pl.ds(s * RPS, RPS)])
    plsc.subcore_barrier()

    @pl.loop(0, NG)
    def _grp(g):
        pltpu.sync_copy(src_hbm.at[w, pl.ds(g * GS, GS)], src_v)
        pltpu.sync_copy(dst_hbm.at[w, pl.ds(g * GS, GS)], dst_v)
        pltpu.async_copy(h_hbm.at[src_v.at[0]], rows_v.at[0], sem0)

        @pl.loop(0, GS, step=2)
        def _edges(j):
            for b in range(2):
                jj = j + b

                @pl.when(jj + 1 < GS)
                def _():
                    pltpu.async_copy(h_hbm.at[src_v.at[jj + 1]],
                                     rows_v.at[(b + 1) % 2], sems[(b + 1) % 2])

                pltpu.make_async_copy(h_hbm.at[src_v.at[jj]],
                                      rows_v.at[b], sems[b]).wait()
                pltpu.sync_copy(rows_v.at[b], acc.at[dst_v.at[jj]], add=True)

    plsc.subcore_barrier()
    pltpu.sync_copy(acc.at[pl.ds(s * RPS, RPS)],
                    out_hbm.at[c, pl.ds(s * RPS, RPS)])


def _aggregate(h, src_p, dst_p):
    kern = pl.kernel(
        _agg_body,
        out_type=jax.ShapeDtypeStruct((NC, NPAD, D), jnp.float32),
        mesh=plsc.VectorSubcoreMesh(core_axis_name="c", subcore_axis_name="s"),
        scratch_types=[
            pltpu.VMEM_SHARED((NPAD, D), jnp.float32),
            pltpu.VMEM((GS, CHUNK), jnp.int32),
            pltpu.VMEM((GS, CHUNK), jnp.int32),
            pltpu.VMEM((2, CHUNK, D), jnp.float32),
            pltpu.SemaphoreType.DMA,
            pltpu.SemaphoreType.DMA,
        ],
    )
    return kern(h, src_p, dst_p)


def _mlp_body(relu_out, p0_ref, p1_ref, h_ref, w1_ref, b1_ref, w2_ref, b2_ref,
              o_ref):
    m = p0_ref[...] + p1_ref[...] - h_ref[...]
    z = jnp.dot(m, w1_ref[...], preferred_element_type=jnp.float32, precision=lax.Precision.HIGHEST)
    z = jnp.maximum(z + b1_ref[...], 0.0)
    z = jnp.dot(z, w2_ref[...], preferred_element_type=jnp.float32, precision=lax.Precision.HIGHEST)
    z = z + b2_ref[...]
    if relu_out:
        z = jnp.maximum(z, 0.0)
    o_ref[...] = z


def _mlp(p0, p1, h, w1, b1, w2, b2, relu_out):
    row = pl.BlockSpec((BLK, D), lambda i: (i, 0))
    full = pl.BlockSpec((D, D), lambda i: (0, 0))
    vec = pl.BlockSpec((1, D), lambda i: (0, 0))
    return pl.pallas_call(
        functools.partial(_mlp_body, relu_out),
        grid=(NBLK,),
        in_specs=[row, row, row, full, vec, full, vec],
        out_specs=row,
        out_shape=jax.ShapeDtypeStruct((NPAD, D), jnp.float32),
    )(p0, p1, h, w1, b1.reshape(1, D), w2, b2.reshape(1, D))


def _final_body(p0_ref, p1_ref, h_ref, w1_ref, b1_ref, w2_ref, b2_ref,
                batch_ref, wfc_ref, bfc_ref, o_ref, pooled):
    i = pl.program_id(0)
    m = p0_ref[...] + p1_ref[...] - h_ref[...]
    z = jnp.dot(m, w1_ref[...], preferred_element_type=jnp.float32, precision=lax.Precision.HIGHEST)
    z = jnp.maximum(z + b1_ref[...], 0.0)
    z = jnp.dot(z, w2_ref[...], preferred_element_type=jnp.float32, precision=lax.Precision.HIGHEST)
    z = z + b2_ref[...]
    oh = (batch_ref[...] ==
          lax.broadcasted_iota(jnp.int32, (BLK, G), 1)).astype(jnp.float32)
    part = lax.dot_general(oh, z, (((0,), (0,)), ((), ())),
                           preferred_element_type=jnp.float32, precision=lax.Precision.HIGHEST)

    @pl.when(i == 0)
    def _():
        pooled[...] = jnp.zeros_like(pooled)

    pooled[...] += part

    @pl.when(i == pl.num_programs(0) - 1)
    def _():
        o_ref[...] = (jnp.dot(pooled[...], wfc_ref[...],
                              preferred_element_type=jnp.float32, precision=lax.Precision.HIGHEST)
                      + bfc_ref[...])


def _final(p0, p1, h, w1, b1, w2, b2, batch2d, wfc, bfc):
    row = pl.BlockSpec((BLK, D), lambda i: (i, 0))
    full = pl.BlockSpec((D, D), lambda i: (0, 0))
    vec = pl.BlockSpec((1, D), lambda i: (0, 0))
    return pl.pallas_call(
        _final_body,
        grid=(NBLK,),
        in_specs=[row, row, row, full, vec, full, vec,
                  pl.BlockSpec((BLK, 1), lambda i: (i, 0)),
                  pl.BlockSpec((D, 1), lambda i: (0, 0)),
                  pl.BlockSpec((1, 1), lambda i: (0, 0))],
        out_specs=pl.BlockSpec((G, 1), lambda i: (0, 0)),
        out_shape=jax.ShapeDtypeStruct((G, 1), jnp.float32),
        scratch_shapes=[pltpu.VMEM((G, D), jnp.float32)],
    )(p0, p1, h, w1, b1.reshape(1, D), w2, b2.reshape(1, D),
      batch2d, wfc, bfc.reshape(1, 1))


def kernel(x, edge_index, batch, W1_0, b1_0, W2_0, b2_0, W1_1, b1_1, W2_1,
           b2_1, W1_2, b1_2, W2_2, b2_2, Wfc, bfc):
    src = edge_index[0]
    dst = edge_index[1]
    pad = EPAD - E
    dst_pad = N + (jnp.arange(pad, dtype=jnp.int32) % (NPAD - N))
    src_p = jnp.concatenate(
        [src, jnp.zeros((pad,), jnp.int32)]).reshape(NW, CPW, CHUNK)
    dst_p = jnp.concatenate(
        [dst, dst_pad]).reshape(NW, CPW, CHUNK)
    batch2d = jnp.concatenate(
        [batch, jnp.full((NPAD - N,), G, jnp.int32)]).reshape(NPAD, 1)

    h = jnp.pad(x.astype(jnp.float32), ((0, NPAD - N), (0, 0)))
    params = [(W1_0, b1_0, W2_0, b2_0), (W1_1, b1_1, W2_1, b2_1),
              (W1_2, b1_2, W2_2, b2_2)]
    for i in range(2):
        w1, b1, w2, b2 = params[i]
        p = _aggregate(h, src_p, dst_p)
        h = _mlp(p[0], p[1], h, w1, b1, w2, b2, relu_out=True)
    w1, b1, w2, b2 = params[2]
    p = _aggregate(h, src_p, dst_p)
    return _final(p[0], p[1], h, w1, b1, w2, b2, batch2d, Wfc, bfc)


# speedup vs baseline: 3.2159x; 3.2159x over previous
import functools

import jax
import jax.numpy as jnp
from jax import lax
from jax.experimental import pallas as pl
from jax.experimental.pallas import tpu as pltpu
from jax.experimental.pallas import tpu_sc as plsc

N, E, D, G = 10000, 320000, 128, 128
NC, NS = 2, 16
NW = NC * NS
CHUNK = 128
CPW = 80
GS = 16
NG = CPW // GS
EPAD = NW * CPW * CHUNK
NCHUNK = E // CHUNK
NPAD = 10240
RPS = NPAD // NS

BLK = 2048
NBLK = NPAD // BLK


def _agg_body(h_hbm, src_hbm, dst_hbm, out_hbm,
              acc, src_v, dst_v, rows_v, sem0, sem1):
    c = lax.axis_index("c")
    s = lax.axis_index("s")
    w = c * NS + s
    sems = (sem0, sem1)
    n_w = jnp.int32(NCHUNK // NW) + (w < NCHUNK % NW).astype(jnp.int32)

    pltpu.sync_copy(h_hbm.at[pl.ds(s * RPS, RPS)], acc.at[pl.ds(s * RPS, RPS)])
    plsc.subcore_barrier()

    @pl.loop(0, NG)
    def _grp(g):
        lim = jnp.minimum(n_w - g * GS, GS)
        pltpu.sync_copy(src_hbm.at[w, pl.ds(g * GS, GS)], src_v)
        pltpu.sync_copy(dst_hbm.at[w, pl.ds(g * GS, GS)], dst_v)

        @pl.when(lim > 0)
        def _():
            pltpu.async_copy(h_hbm.at[src_v.at[0]], rows_v.at[0], sem0)

        @pl.loop(0, GS, step=2)
        def _edges(j):
            for b in range(2):
                jj = j + b

                @pl.when(jj + 1 < lim)
                def _():
                    pltpu.async_copy(h_hbm.at[src_v.at[jj + 1]],
                                     rows_v.at[(b + 1) % 2], sems[(b + 1) % 2])

                @pl.when(jj < lim)
                def _():
                    pltpu.make_async_copy(h_hbm.at[src_v.at[jj]],
                                          rows_v.at[b], sems[b]).wait()
                    pltpu.sync_copy(rows_v.at[b], acc.at[dst_v.at[jj]],
                                    add=True)

    plsc.subcore_barrier()
    pltpu.sync_copy(acc.at[pl.ds(s * RPS, RPS)],
                    out_hbm.at[c, pl.ds(s * RPS, RPS)])


def _aggregate(h, src_p, dst_p):
    kern = pl.kernel(
        _agg_body,
        out_type=jax.ShapeDtypeStruct((NC, NPAD, D), jnp.float32),
        mesh=plsc.VectorSubcoreMesh(core_axis_name="c", subcore_axis_name="s"),
        scratch_types=[
            pltpu.VMEM_SHARED((NPAD, D), jnp.float32),
            pltpu.VMEM((GS, CHUNK), jnp.int32),
            pltpu.VMEM((GS, CHUNK), jnp.int32),
            pltpu.VMEM((2, CHUNK, D), jnp.float32),
            pltpu.SemaphoreType.DMA,
            pltpu.SemaphoreType.DMA,
        ],
    )
    return kern(h, src_p, dst_p)


def _mlp_body(relu_out, p0_ref, p1_ref, h_ref, w1_ref, b1_ref, w2_ref, b2_ref,
              o_ref):
    m = p0_ref[...] + p1_ref[...] - h_ref[...]
    z = jnp.dot(m, w1_ref[...], preferred_element_type=jnp.float32, precision=lax.Precision.HIGHEST)
    z = jnp.maximum(z + b1_ref[...], 0.0)
    z = jnp.dot(z, w2_ref[...], preferred_element_type=jnp.float32, precision=lax.Precision.HIGHEST)
    z = z + b2_ref[...]
    if relu_out:
        z = jnp.maximum(z, 0.0)
    o_ref[...] = z


def _mlp(p0, p1, h, w1, b1, w2, b2, relu_out):
    row = pl.BlockSpec((BLK, D), lambda i: (i, 0))
    full = pl.BlockSpec((D, D), lambda i: (0, 0))
    vec = pl.BlockSpec((1, D), lambda i: (0, 0))
    return pl.pallas_call(
        functools.partial(_mlp_body, relu_out),
        grid=(NBLK,),
        in_specs=[row, row, row, full, vec, full, vec],
        out_specs=row,
        out_shape=jax.ShapeDtypeStruct((NPAD, D), jnp.float32),
    )(p0, p1, h, w1, b1.reshape(1, D), w2, b2.reshape(1, D))


def _final_body(p0_ref, p1_ref, h_ref, w1_ref, b1_ref, w2_ref, b2_ref,
                batch_ref, wfc_ref, bfc_ref, o_ref, pooled):
    i = pl.program_id(0)
    m = p0_ref[...] + p1_ref[...] - h_ref[...]
    z = jnp.dot(m, w1_ref[...], preferred_element_type=jnp.float32, precision=lax.Precision.HIGHEST)
    z = jnp.maximum(z + b1_ref[...], 0.0)
    z = jnp.dot(z, w2_ref[...], preferred_element_type=jnp.float32, precision=lax.Precision.HIGHEST)
    z = z + b2_ref[...]
    oh = (batch_ref[...] ==
          lax.broadcasted_iota(jnp.int32, (BLK, G), 1)).astype(jnp.float32)
    part = lax.dot_general(oh, z, (((0,), (0,)), ((), ())),
                           preferred_element_type=jnp.float32, precision=lax.Precision.HIGHEST)

    @pl.when(i == 0)
    def _():
        pooled[...] = jnp.zeros_like(pooled)

    pooled[...] += part

    @pl.when(i == pl.num_programs(0) - 1)
    def _():
        o_ref[...] = (jnp.dot(pooled[...], wfc_ref[...],
                              preferred_element_type=jnp.float32, precision=lax.Precision.HIGHEST)
                      + bfc_ref[...])


def _final(p0, p1, h, w1, b1, w2, b2, batch2d, wfc, bfc):
    row = pl.BlockSpec((BLK, D), lambda i: (i, 0))
    full = pl.BlockSpec((D, D), lambda i: (0, 0))
    vec = pl.BlockSpec((1, D), lambda i: (0, 0))
    return pl.pallas_call(
        _final_body,
        grid=(NBLK,),
        in_specs=[row, row, row, full, vec, full, vec,
                  pl.BlockSpec((BLK, 1), lambda i: (i, 0)),
                  pl.BlockSpec((D, 1), lambda i: (0, 0)),
                  pl.BlockSpec((1, 1), lambda i: (0, 0))],
        out_specs=pl.BlockSpec((G, 1), lambda i: (0, 0)),
        out_shape=jax.ShapeDtypeStruct((G, 1), jnp.float32),
        scratch_shapes=[pltpu.VMEM((G, D), jnp.float32)],
    )(p0, p1, h, w1, b1.reshape(1, D), w2, b2.reshape(1, D),
      batch2d, wfc, bfc.reshape(1, 1))


def kernel(x, edge_index, batch, W1_0, b1_0, W2_0, b2_0, W1_1, b1_1, W2_1,
           b2_1, W1_2, b1_2, W2_2, b2_2, Wfc, bfc):
    src = edge_index[0]
    dst = edge_index[1]
    pad = EPAD - E
    src_p = jnp.concatenate(
        [src, jnp.zeros((pad,), jnp.int32)]
    ).reshape(CPW, NW, CHUNK).transpose(1, 0, 2)
    dst_p = jnp.concatenate(
        [dst, jnp.full((pad,), N, jnp.int32)]
    ).reshape(CPW, NW, CHUNK).transpose(1, 0, 2)
    batch2d = jnp.concatenate(
        [batch, jnp.full((NPAD - N,), G, jnp.int32)]).reshape(NPAD, 1)

    h = jnp.pad(x.astype(jnp.float32), ((0, NPAD - N), (0, 0)))
    params = [(W1_0, b1_0, W2_0, b2_0), (W1_1, b1_1, W2_1, b2_1),
              (W1_2, b1_2, W2_2, b2_2)]
    for i in range(2):
        w1, b1, w2, b2 = params[i]
        p = _aggregate(h, src_p, dst_p)
        h = _mlp(p[0], p[1], h, w1, b1, w2, b2, relu_out=True)
    w1, b1, w2, b2 = params[2]
    p = _aggregate(h, src_p, dst_p)
    return _final(p[0], p[1], h, w1, b1, w2, b2, batch2d, Wfc, bfc)


# speedup vs baseline: 3.3984x; 1.0568x over previous
import functools

import jax
import jax.numpy as jnp
from jax import lax
from jax.experimental import pallas as pl
from jax.experimental.pallas import tpu as pltpu
from jax.experimental.pallas import tpu_sc as plsc

N, E, D, G = 10000, 320000, 128, 128
NC, NS = 2, 16
NW = NC * NS
CHUNK = 128
CPW = 80
GS = 40
NG = CPW // GS
EPAD = NW * CPW * CHUNK
NCHUNK = E // CHUNK
NPAD = 10240
RPS = NPAD // NS

BLK = 1024
NBLK = NPAD // BLK


def _agg_body(h_hbm, src_hbm, dst_hbm, out_hbm,
              acc, src_v, dst_v, rows_v, sem0, sem1, isem):
    c = lax.axis_index("c")
    s = lax.axis_index("s")
    w = c * NS + s
    sems = (sem0, sem1)
    n_w = jnp.int32(NCHUNK // NW) + (w < NCHUNK % NW).astype(jnp.int32)

    init_cp = pltpu.async_copy(h_hbm.at[pl.ds(s * RPS, RPS)],
                               acc.at[pl.ds(s * RPS, RPS)], isem)
    pltpu.sync_copy(src_hbm.at[w, pl.ds(0, GS)], src_v)
    pltpu.sync_copy(dst_hbm.at[w, pl.ds(0, GS)], dst_v)
    init_cp.wait()
    plsc.subcore_barrier()

    @pl.loop(0, NG)
    def _grp(g):
        lim = jnp.minimum(n_w - g * GS, GS)

        @pl.when(g > 0)
        def _():
            pltpu.sync_copy(src_hbm.at[w, pl.ds(g * GS, GS)], src_v)
            pltpu.sync_copy(dst_hbm.at[w, pl.ds(g * GS, GS)], dst_v)

        @pl.when(lim > 0)
        def _():
            pltpu.async_copy(h_hbm.at[src_v.at[0]], rows_v.at[0], sem0)

        @pl.loop(0, GS, step=2)
        def _edges(j):
            for b in range(2):
                jj = j + b

                @pl.when(jj + 1 < lim)
                def _():
                    pltpu.async_copy(h_hbm.at[src_v.at[jj + 1]],
                                     rows_v.at[(b + 1) % 2], sems[(b + 1) % 2])

                @pl.when(jj < lim)
                def _():
                    pltpu.make_async_copy(h_hbm.at[src_v.at[jj]],
                                          rows_v.at[b], sems[b]).wait()
                    pltpu.sync_copy(rows_v.at[b], acc.at[dst_v.at[jj]],
                                    add=True)

    plsc.subcore_barrier()
    pltpu.sync_copy(acc.at[pl.ds(s * RPS, RPS)],
                    out_hbm.at[c, pl.ds(s * RPS, RPS)])


def _aggregate(h, src_p, dst_p):
    kern = pl.kernel(
        _agg_body,
        out_type=jax.ShapeDtypeStruct((NC, NPAD, D), jnp.float32),
        mesh=plsc.VectorSubcoreMesh(core_axis_name="c", subcore_axis_name="s"),
        scratch_types=[
            pltpu.VMEM_SHARED((NPAD, D), jnp.float32),
            pltpu.VMEM((GS, CHUNK), jnp.int32),
            pltpu.VMEM((GS, CHUNK), jnp.int32),
            pltpu.VMEM((2, CHUNK, D), jnp.float32),
            pltpu.SemaphoreType.DMA,
            pltpu.SemaphoreType.DMA,
            pltpu.SemaphoreType.DMA,
        ],
    )
    return kern(h, src_p, dst_p)


def _mlp_body(relu_out, p0_ref, p1_ref, h_ref, w1_ref, b1_ref, w2_ref, b2_ref,
              o_ref):
    m = p0_ref[...] + p1_ref[...] - h_ref[...]
    z = jnp.dot(m, w1_ref[...], preferred_element_type=jnp.float32, precision=lax.Precision.HIGHEST)
    z = jnp.maximum(z + b1_ref[...], 0.0)
    z = jnp.dot(z, w2_ref[...], preferred_element_type=jnp.float32, precision=lax.Precision.HIGHEST)
    z = z + b2_ref[...]
    if relu_out:
        z = jnp.maximum(z, 0.0)
    o_ref[...] = z


def _mlp(p0, p1, h, w1, b1, w2, b2, relu_out):
    row = pl.BlockSpec((BLK, D), lambda i: (i, 0))
    full = pl.BlockSpec((D, D), lambda i: (0, 0))
    vec = pl.BlockSpec((1, D), lambda i: (0, 0))
    return pl.pallas_call(
        functools.partial(_mlp_body, relu_out),
        grid=(NBLK,),
        in_specs=[row, row, row, full, vec, full, vec],
        out_specs=row,
        out_shape=jax.ShapeDtypeStruct((NPAD, D), jnp.float32),
    )(p0, p1, h, w1, b1.reshape(1, D), w2, b2.reshape(1, D))


def _final_body(p0_ref, p1_ref, h_ref, w1_ref, b1_ref, w2_ref, b2_ref,
                batch_ref, wfc_ref, bfc_ref, o_ref, pooled):
    i = pl.program_id(0)
    m = p0_ref[...] + p1_ref[...] - h_ref[...]
    z = jnp.dot(m, w1_ref[...], preferred_element_type=jnp.float32, precision=lax.Precision.HIGHEST)
    z = jnp.maximum(z + b1_ref[...], 0.0)
    z = jnp.dot(z, w2_ref[...], preferred_element_type=jnp.float32, precision=lax.Precision.HIGHEST)
    z = z + b2_ref[...]
    oh = (batch_ref[...] ==
          lax.broadcasted_iota(jnp.int32, (BLK, G), 1)).astype(jnp.float32)
    part = lax.dot_general(oh, z, (((0,), (0,)), ((), ())),
                           preferred_element_type=jnp.float32, precision=lax.Precision.HIGHEST)

    @pl.when(i == 0)
    def _():
        pooled[...] = jnp.zeros_like(pooled)

    pooled[...] += part

    @pl.when(i == pl.num_programs(0) - 1)
    def _():
        o_ref[...] = (jnp.dot(pooled[...], wfc_ref[...],
                              preferred_element_type=jnp.float32, precision=lax.Precision.HIGHEST)
                      + bfc_ref[...])


def _final(p0, p1, h, w1, b1, w2, b2, batch2d, wfc, bfc):
    row = pl.BlockSpec((BLK, D), lambda i: (i, 0))
    full = pl.BlockSpec((D, D), lambda i: (0, 0))
    vec = pl.BlockSpec((1, D), lambda i: (0, 0))
    return pl.pallas_call(
        _final_body,
        grid=(NBLK,),
        in_specs=[row, row, row, full, vec, full, vec,
                  pl.BlockSpec((BLK, 1), lambda i: (i, 0)),
                  pl.BlockSpec((D, 1), lambda i: (0, 0)),
                  pl.BlockSpec((1, 1), lambda i: (0, 0))],
        out_specs=pl.BlockSpec((G, 1), lambda i: (0, 0)),
        out_shape=jax.ShapeDtypeStruct((G, 1), jnp.float32),
        scratch_shapes=[pltpu.VMEM((G, D), jnp.float32)],
    )(p0, p1, h, w1, b1.reshape(1, D), w2, b2.reshape(1, D),
      batch2d, wfc, bfc.reshape(1, 1))


def kernel(x, edge_index, batch, W1_0, b1_0, W2_0, b2_0, W1_1, b1_1, W2_1,
           b2_1, W1_2, b1_2, W2_2, b2_2, Wfc, bfc):
    src = edge_index[0]
    dst = edge_index[1]
    pad = EPAD - E
    src_p = jnp.concatenate(
        [src, jnp.zeros((pad,), jnp.int32)]
    ).reshape(CPW, NW, CHUNK).transpose(1, 0, 2)
    dst_p = jnp.concatenate(
        [dst, jnp.full((pad,), N, jnp.int32)]
    ).reshape(CPW, NW, CHUNK).transpose(1, 0, 2)
    batch2d = jnp.concatenate(
        [batch, jnp.full((NPAD - N,), G, jnp.int32)]).reshape(NPAD, 1)

    h = jnp.pad(x.astype(jnp.float32), ((0, NPAD - N), (0, 0)))
    params = [(W1_0, b1_0, W2_0, b2_0), (W1_1, b1_1, W2_1, b2_1),
              (W1_2, b1_2, W2_2, b2_2)]
    for i in range(2):
        w1, b1, w2, b2 = params[i]
        p = _aggregate(h, src_p, dst_p)
        h = _mlp(p[0], p[1], h, w1, b1, w2, b2, relu_out=True)
    w1, b1, w2, b2 = params[2]
    p = _aggregate(h, src_p, dst_p)
    return _final(p[0], p[1], h, w1, b1, w2, b2, batch2d, Wfc, bfc)


# speedup vs baseline: 3.6267x; 1.0672x over previous
import functools

import jax
import jax.numpy as jnp
from jax import lax
from jax.experimental import pallas as pl
from jax.experimental.pallas import tpu as pltpu
from jax.experimental.pallas import tpu_sc as plsc

N, E, D, G = 10000, 320000, 128, 128
NC, NS = 2, 16
NW = NC * NS
CHUNK = 128
CPW = 80
GS = 40
NG = CPW // GS
EPAD = NW * CPW * CHUNK
NCHUNK = E // CHUNK
NPAD = 10240
RPS = NPAD // NS

BLK = 1024
NBLK = NPAD // BLK


def _agg_body(h_hbm, src_hbm, dst_hbm, out_hbm,
              acc, src_v, dst_v, rows_v, sem0, sem1, isem):
    c = lax.axis_index("c")
    s = lax.axis_index("s")
    w = c * NS + s
    sems = (sem0, sem1)
    n_w = jnp.int32(NCHUNK // NW) + (w < NCHUNK % NW).astype(jnp.int32)

    init_cp = pltpu.async_copy(h_hbm.at[pl.ds(s * RPS, RPS)],
                               acc.at[pl.ds(s * RPS, RPS)], isem)
    pltpu.sync_copy(src_hbm.at[w, pl.ds(0, GS)], src_v)
    pltpu.sync_copy(dst_hbm.at[w, pl.ds(0, GS)], dst_v)
    init_cp.wait()
    plsc.subcore_barrier()

    @pl.loop(0, NG)
    def _grp(g):
        lim = jnp.minimum(n_w - g * GS, GS)

        @pl.when(g > 0)
        def _():
            pltpu.sync_copy(src_hbm.at[w, pl.ds(g * GS, GS)], src_v)
            pltpu.sync_copy(dst_hbm.at[w, pl.ds(g * GS, GS)], dst_v)

        @pl.when(lim > 0)
        def _():
            pltpu.async_copy(h_hbm.at[src_v.at[0]], rows_v.at[0], sem0)

        @pl.loop(0, GS, step=2)
        def _edges(j):
            for b in range(2):
                jj = j + b

                @pl.when(jj + 1 < lim)
                def _():
                    pltpu.async_copy(h_hbm.at[src_v.at[jj + 1]],
                                     rows_v.at[(b + 1) % 2], sems[(b + 1) % 2])

                @pl.when(jj < lim)
                def _():
                    pltpu.make_async_copy(h_hbm.at[src_v.at[jj]],
                                          rows_v.at[b], sems[b]).wait()
                    pltpu.sync_copy(rows_v.at[b], acc.at[dst_v.at[jj]],
                                    add=True)

    plsc.subcore_barrier()
    pltpu.sync_copy(acc.at[pl.ds(s * RPS, RPS)],
                    out_hbm.at[c, pl.ds(s * RPS, RPS)])


def _aggregate(h, src_p, dst_p):
    kern = pl.kernel(
        _agg_body,
        out_type=jax.ShapeDtypeStruct((NC, NPAD, D), jnp.float32),
        mesh=plsc.VectorSubcoreMesh(core_axis_name="c", subcore_axis_name="s"),
        scratch_types=[
            pltpu.VMEM_SHARED((NPAD, D), jnp.float32),
            pltpu.VMEM((GS, CHUNK), jnp.int32),
            pltpu.VMEM((GS, CHUNK), jnp.int32),
            pltpu.VMEM((2, CHUNK, D), jnp.float32),
            pltpu.SemaphoreType.DMA,
            pltpu.SemaphoreType.DMA,
            pltpu.SemaphoreType.DMA,
        ],
    )
    return kern(h, src_p, dst_p)


def _mlp_body(relu_out, p0_ref, p1_ref, h_ref, w1_ref, b1_ref, w2_ref, b2_ref,
              o_ref):
    m = p0_ref[...] + p1_ref[...] - h_ref[...]
    z = jnp.dot(m, w1_ref[...], preferred_element_type=jnp.float32)
    z = jnp.maximum(z + b1_ref[...], 0.0)
    z = jnp.dot(z, w2_ref[...], preferred_element_type=jnp.float32)
    z = z + b2_ref[...]
    if relu_out:
        z = jnp.maximum(z, 0.0)
    o_ref[...] = z


def _mlp(p0, p1, h, w1, b1, w2, b2, relu_out):
    row = pl.BlockSpec((BLK, D), lambda i: (i, 0))
    full = pl.BlockSpec((D, D), lambda i: (0, 0))
    vec = pl.BlockSpec((1, D), lambda i: (0, 0))
    return pl.pallas_call(
        functools.partial(_mlp_body, relu_out),
        grid=(NBLK,),
        in_specs=[row, row, row, full, vec, full, vec],
        out_specs=row,
        out_shape=jax.ShapeDtypeStruct((NPAD, D), jnp.float32),
    )(p0, p1, h, w1, b1.reshape(1, D), w2, b2.reshape(1, D))


def _final_body(p0_ref, p1_ref, h_ref, w1_ref, b1_ref, w2_ref, b2_ref,
                batch_ref, wfc_ref, bfc_ref, o_ref, pooled):
    i = pl.program_id(0)
    m = p0_ref[...] + p1_ref[...] - h_ref[...]
    z = jnp.dot(m, w1_ref[...], preferred_element_type=jnp.float32)
    z = jnp.maximum(z + b1_ref[...], 0.0)
    z = jnp.dot(z, w2_ref[...], preferred_element_type=jnp.float32)
    z = z + b2_ref[...]
    oh = (batch_ref[...] ==
          lax.broadcasted_iota(jnp.int32, (BLK, G), 1)).astype(jnp.float32)
    part = lax.dot_general(oh, z, (((0,), (0,)), ((), ())),
                           preferred_element_type=jnp.float32, precision=lax.Precision.HIGHEST)

    @pl.when(i == 0)
    def _():
        pooled[...] = jnp.zeros_like(pooled)

    pooled[...] += part

    @pl.when(i == pl.num_programs(0) - 1)
    def _():
        o_ref[...] = (jnp.dot(pooled[...], wfc_ref[...],
                              preferred_element_type=jnp.float32)
                      + bfc_ref[...])


def _final(p0, p1, h, w1, b1, w2, b2, batch2d, wfc, bfc):
    row = pl.BlockSpec((BLK, D), lambda i: (i, 0))
    full = pl.BlockSpec((D, D), lambda i: (0, 0))
    vec = pl.BlockSpec((1, D), lambda i: (0, 0))
    return pl.pallas_call(
        _final_body,
        grid=(NBLK,),
        in_specs=[row, row, row, full, vec, full, vec,
                  pl.BlockSpec((BLK, 1), lambda i: (i, 0)),
                  pl.BlockSpec((D, 1), lambda i: (0, 0)),
                  pl.BlockSpec((1, 1), lambda i: (0, 0))],
        out_specs=pl.BlockSpec((G, 1), lambda i: (0, 0)),
        out_shape=jax.ShapeDtypeStruct((G, 1), jnp.float32),
        scratch_shapes=[pltpu.VMEM((G, D), jnp.float32)],
    )(p0, p1, h, w1, b1.reshape(1, D), w2, b2.reshape(1, D),
      batch2d, wfc, bfc.reshape(1, 1))


def kernel(x, edge_index, batch, W1_0, b1_0, W2_0, b2_0, W1_1, b1_1, W2_1,
           b2_1, W1_2, b1_2, W2_2, b2_2, Wfc, bfc):
    src = edge_index[0]
    dst = edge_index[1]
    pad = EPAD - E
    src_p = jnp.concatenate(
        [src, jnp.zeros((pad,), jnp.int32)]
    ).reshape(CPW, NW, CHUNK).transpose(1, 0, 2)
    dst_p = jnp.concatenate(
        [dst, jnp.full((pad,), N, jnp.int32)]
    ).reshape(CPW, NW, CHUNK).transpose(1, 0, 2)
    batch2d = jnp.concatenate(
        [batch, jnp.full((NPAD - N,), G, jnp.int32)]).reshape(NPAD, 1)

    h = jnp.pad(x.astype(jnp.float32), ((0, NPAD - N), (0, 0)))
    params = [(W1_0, b1_0, W2_0, b2_0), (W1_1, b1_1, W2_1, b2_1),
              (W1_2, b1_2, W2_2, b2_2)]
    for i in range(2):
        w1, b1, w2, b2 = params[i]
        p = _aggregate(h, src_p, dst_p)
        h = _mlp(p[0], p[1], h, w1, b1, w2, b2, relu_out=True)
    w1, b1, w2, b2 = params[2]
    p = _aggregate(h, src_p, dst_p)
    return _final(p[0], p[1], h, w1, b1, w2, b2, batch2d, Wfc, bfc)
